# baseline (speedup 1.0000x reference)
import functools

import jax
import jax.numpy as jnp
from jax import lax
from jax.experimental import pallas as pl
from jax.experimental.pallas import tpu as pltpu
from jax.experimental.pallas import tpu_sc as plsc

D_MODEL = 128
SEQ = 200
G = 80
NUM_WORKERS = 32
LANES = 16
PE_EXT = SEQ + G - 40


def _build_kernel(n_tokens):
    per_w = n_tokens // NUM_WORKERS
    nchunks = per_w // G

    mesh = plsc.VectorSubcoreMesh(core_axis_name="c", subcore_axis_name="s")

    @functools.partial(
        pl.kernel,
        out_type=jax.ShapeDtypeStruct((n_tokens, D_MODEL), jnp.float32),
        mesh=mesh,
        scratch_types=[
            pltpu.VMEM((per_w,), jnp.int32),
            pltpu.VMEM((PE_EXT, D_MODEL), jnp.float32),
            pltpu.VMEM((G, D_MODEL), jnp.float32),
            pltpu.VMEM((G, D_MODEL), jnp.float32),
            pltpu.VMEM((G, D_MODEL), jnp.float32),
            pltpu.VMEM((G, D_MODEL), jnp.float32),
            pltpu.SemaphoreType.DMA,
            pltpu.SemaphoreType.DMA,
            pltpu.SemaphoreType.DMA,
            pltpu.SemaphoreType.DMA,
        ],
    )
    def embed(tok_hbm, table_hbm, pe_hbm, out_hbm,
              tok_v, pe_v, g0, g1, o0, o1, sg0, sg1, sw0, sw1):
        wid = lax.axis_index("s") * 2 + lax.axis_index("c")
        base = wid * per_w

        pltpu.sync_copy(tok_hbm.at[pl.ds(base, per_w)], tok_v)
        pltpu.sync_copy(pe_hbm, pe_v.at[pl.ds(0, SEQ)])
        pltpu.sync_copy(pe_hbm.at[pl.ds(0, PE_EXT - SEQ)],
                        pe_v.at[pl.ds(SEQ, PE_EXT - SEQ)])

        gs = (g0, g1)
        os_ = (o0, o1)
        sgs = (sg0, sg1)
        sws = (sw0, sw1)

        def start_gather(c, s):
            pltpu.async_copy(
                table_hbm.at[tok_v.at[pl.ds(c * G, G)]], gs[s], sgs[s])

        def wait_gather(s):
            pltpu.make_async_copy(
                table_hbm.at[tok_v.at[pl.ds(0, G)]], gs[s], sgs[s]).wait()

        def start_write(c, s):
            pltpu.async_copy(
                os_[s], out_hbm.at[pl.ds(base + c * G, G)], sws[s])

        def wait_write(s):
            pltpu.make_async_copy(
                os_[s], out_hbm.at[pl.ds(base, G)], sws[s]).wait()

        def compute(c, s):
            pb = (c * G) % SEQ
            gv = gs[s]
            ov = os_[s]

            @plsc.parallel_loop(0, G, unroll=4)
            def _row(t):
                pr = pb + t
                slices = [pl.ds(j * LANES, LANES) for j in range(D_MODEL // LANES)]
                gvals = [gv[t, sl] for sl in slices]
                pvals = [pe_v[pr, sl] for sl in slices]
                for sl, gval, pval in zip(slices, gvals, pvals):
                    ov[t, sl] = gval + pval

        start_gather(0, 0)
        start_gather(1, 1)
        for s in (0, 1):
            wait_gather(s)
            compute(s, s)
            start_write(s, s)
            start_gather(s + 2, s)

        @pl.loop(2, nchunks - 2, step=2)
        def _steady(c0):
            for s in (0, 1):
                c = c0 + s
                wait_gather(s)
                wait_write(s)
                compute(c, s)
                start_write(c, s)
                start_gather(c + 2, s)

        for s in (0, 1):
            c = nchunks - 2 + s
            wait_gather(s)
            wait_write(s)
            compute(c, s)
            start_write(c, s)
        for s in (0, 1):
            wait_write(s)

    return embed


def kernel(tokens, table, PE):
    batch, seq = tokens.shape
    n_tokens = batch * seq
    out = _build_kernel(n_tokens)(
        tokens.reshape(n_tokens), table, PE[:seq])
    return out.reshape(batch, seq, D_MODEL)


# speedup vs baseline: 1.0024x; 1.0024x over previous
import functools

import jax
import jax.numpy as jnp
from jax import lax
from jax.experimental import pallas as pl
from jax.experimental.pallas import tpu as pltpu
from jax.experimental.pallas import tpu_sc as plsc

D_MODEL = 128
SEQ = 200
G = 80
NUM_WORKERS = 32
LANES = 16
PE_EXT = SEQ + G - 40


def _build_kernel(n_tokens):
    per_w = n_tokens // NUM_WORKERS
    nchunks = per_w // G

    mesh = plsc.VectorSubcoreMesh(core_axis_name="c", subcore_axis_name="s")

    @functools.partial(
        pl.kernel,
        out_type=jax.ShapeDtypeStruct((n_tokens, D_MODEL), jnp.float32),
        mesh=mesh,
        scratch_types=[
            pltpu.VMEM((per_w,), jnp.int32),
            pltpu.VMEM((PE_EXT, D_MODEL), jnp.float32),
            pltpu.VMEM((G, D_MODEL), jnp.float32),
            pltpu.VMEM((G, D_MODEL), jnp.float32),
            pltpu.VMEM((G, D_MODEL), jnp.float32),
            pltpu.VMEM((G, D_MODEL), jnp.float32),
            pltpu.SemaphoreType.DMA,
            pltpu.SemaphoreType.DMA,
            pltpu.SemaphoreType.DMA,
            pltpu.SemaphoreType.DMA,
        ],
    )
    def embed(tok_hbm, table_hbm, pe_hbm, out_hbm,
              tok_v, pe_v, g0, g1, o0, o1, sg0, sg1, sw0, sw1):
        wid = lax.axis_index("s") * 2 + lax.axis_index("c")
        base = wid * per_w

        pltpu.sync_copy(tok_hbm.at[pl.ds(base, per_w)], tok_v)
        pltpu.sync_copy(pe_hbm, pe_v.at[pl.ds(0, SEQ)])
        pltpu.sync_copy(pe_hbm.at[pl.ds(0, PE_EXT - SEQ)],
                        pe_v.at[pl.ds(SEQ, PE_EXT - SEQ)])

        gs = (g0, g1)
        os_ = (o0, o1)
        sgs = (sg0, sg1)
        sws = (sw0, sw1)

        def start_gather(c, s):
            pltpu.async_copy(
                table_hbm.at[tok_v.at[pl.ds(c * G, G)]], gs[s], sgs[s])

        def wait_gather(s):
            pltpu.make_async_copy(
                table_hbm.at[tok_v.at[pl.ds(0, G)]], gs[s], sgs[s]).wait()

        def start_write(c, s):
            pltpu.async_copy(
                os_[s], out_hbm.at[pl.ds(base + c * G, G)], sws[s])

        def wait_write(s):
            pltpu.make_async_copy(
                os_[s], out_hbm.at[pl.ds(base, G)], sws[s]).wait()

        def compute(c, s):
            pb = (c * G) % SEQ
            gv = gs[s]
            ov = os_[s]

            @plsc.parallel_loop(0, G, unroll=2)
            def _row(t):
                pr = pb + t
                slices = [pl.ds(j * LANES, LANES) for j in range(D_MODEL // LANES)]
                gvals = [gv[t, sl] for sl in slices]
                pvals = [pe_v[pr, sl] for sl in slices]
                for sl, gval, pval in zip(slices, gvals, pvals):
                    ov[t, sl] = gval + pval

        start_gather(0, 0)
        start_gather(1, 1)
        for s in (0, 1):
            wait_gather(s)
            compute(s, s)
            start_write(s, s)
            start_gather(s + 2, s)

        @pl.loop(2, nchunks - 2, step=2)
        def _steady(c0):
            for s in (0, 1):
                c = c0 + s
                wait_gather(s)
                wait_write(s)
                compute(c, s)
                start_write(c, s)
                start_gather(c + 2, s)

        for s in (0, 1):
            c = nchunks - 2 + s
            wait_gather(s)
            wait_write(s)
            compute(c, s)
            start_write(c, s)
        for s in (0, 1):
            wait_write(s)

    return embed


def kernel(tokens, table, PE):
    batch, seq = tokens.shape
    n_tokens = batch * seq
    out = _build_kernel(n_tokens)(
        tokens.reshape(n_tokens), table, PE[:seq])
    return out.reshape(batch, seq, D_MODEL)


# speedup vs baseline: 1.0599x; 1.0574x over previous
import functools

import jax
import jax.numpy as jnp
from jax import lax
from jax.experimental import pallas as pl
from jax.experimental.pallas import tpu as pltpu
from jax.experimental.pallas import tpu_sc as plsc

D_MODEL = 128
SEQ = 200
G = 80
NUM_WORKERS = 32
LANES = 16
PE_EXT = SEQ + G - 40
GB = 4
OB = 2


def _build_kernel(n_tokens):
    per_w = n_tokens // NUM_WORKERS
    nchunks = per_w // G
    assert nchunks % GB == 0 and nchunks >= 2 * GB

    mesh = plsc.VectorSubcoreMesh(core_axis_name="c", subcore_axis_name="s")

    @functools.partial(
        pl.kernel,
        out_type=jax.ShapeDtypeStruct((n_tokens, D_MODEL), jnp.float32),
        mesh=mesh,
        scratch_types=[
            pltpu.VMEM((per_w,), jnp.int32),
            pltpu.VMEM((PE_EXT, D_MODEL), jnp.float32),
        ]
        + [pltpu.VMEM((G, D_MODEL), jnp.float32)] * (GB + OB)
        + [pltpu.SemaphoreType.DMA] * (GB + OB),
    )
    def embed(tok_hbm, table_hbm, pe_hbm, out_hbm, tok_v, pe_v, *bufs):
        gs = bufs[:GB]
        os_ = bufs[GB:GB + OB]
        sgs = bufs[GB + OB:2 * GB + OB]
        sws = bufs[2 * GB + OB:]

        wid = lax.axis_index("s") * 2 + lax.axis_index("c")
        base = wid * per_w

        pltpu.sync_copy(tok_hbm.at[pl.ds(base, per_w)], tok_v)
        pltpu.sync_copy(pe_hbm, pe_v.at[pl.ds(0, SEQ)])
        pltpu.sync_copy(pe_hbm.at[pl.ds(0, PE_EXT - SEQ)],
                        pe_v.at[pl.ds(SEQ, PE_EXT - SEQ)])

        def start_gather(c, s):
            pltpu.async_copy(
                table_hbm.at[tok_v.at[pl.ds(c * G, G)]], gs[s], sgs[s])

        def wait_gather(s):
            pltpu.make_async_copy(
                table_hbm.at[tok_v.at[pl.ds(0, G)]], gs[s], sgs[s]).wait()

        def start_write(c, s):
            pltpu.async_copy(
                os_[s], out_hbm.at[pl.ds(base + c * G, G)], sws[s])

        def wait_write(s):
            pltpu.make_async_copy(
                os_[s], out_hbm.at[pl.ds(base, G)], sws[s]).wait()

        def compute(c, s, so):
            pb = (c * G) % SEQ
            gv = gs[s]
            ov = os_[so]

            @plsc.parallel_loop(0, G, unroll=2)
            def _row(t):
                pr = pb + t
                slices = [pl.ds(j * LANES, LANES) for j in range(D_MODEL // LANES)]
                gvals = [gv[t, sl] for sl in slices]
                pvals = [pe_v[pr, sl] for sl in slices]
                for sl, gval, pval in zip(slices, gvals, pvals):
                    ov[t, sl] = gval + pval

        for c in range(GB):
            start_gather(c, c)
        for c in range(GB):
            wait_gather(c)
            if c >= OB:
                wait_write(c % OB)
            compute(c, c, c % OB)
            start_write(c, c % OB)
            start_gather(c + GB, c)

        @pl.loop(GB, nchunks - GB, step=GB)
        def _steady(c0):
            for k in range(GB):
                c = c0 + k
                wait_gather(k)
                wait_write(k % OB)
                compute(c, k, k % OB)
                start_write(c, k % OB)
                start_gather(c + GB, k)

        for k in range(GB):
            c = nchunks - GB + k
            wait_gather(k)
            wait_write(k % OB)
            compute(c, k, k % OB)
            start_write(c, k % OB)
        for s in range(OB):
            wait_write(s)

    return embed


def kernel(tokens, table, PE):
    batch, seq = tokens.shape
    n_tokens = batch * seq
    out = _build_kernel(n_tokens)(
        tokens.reshape(n_tokens), table, PE[:seq])
    return out.reshape(batch, seq, D_MODEL)


# speedup vs baseline: 1.6944x; 1.5986x over previous
import functools

import jax
import jax.numpy as jnp
from jax import lax
from jax.experimental import pallas as pl
from jax.experimental.pallas import tpu as pltpu
from jax.experimental.pallas import tpu_sc as plsc

D_MODEL = 128
SEQ = 200
G = 80
NUM_WORKERS = 32
LANES = 16
PE_EXT = SEQ + G - 40
GB = 4
OB = 2
TBL_PAD = 1024


def _build_kernel(n_tokens):
    per_w = n_tokens // NUM_WORKERS
    nchunks = per_w // G
    assert nchunks % GB == 0 and nchunks >= 2 * GB

    mesh = plsc.VectorSubcoreMesh(core_axis_name="c", subcore_axis_name="s")

    @functools.partial(
        pl.kernel,
        out_type=jax.ShapeDtypeStruct((n_tokens, D_MODEL), jnp.float32),
        mesh=mesh,
        scratch_types=[
            pltpu.VMEM((per_w,), jnp.int32),
            pltpu.VMEM((PE_EXT, D_MODEL), jnp.float32),
            pltpu.VMEM_SHARED((TBL_PAD, D_MODEL), jnp.float32),
        ]
        + [pltpu.VMEM((G, D_MODEL), jnp.float32)] * (GB + OB)
        + [pltpu.SemaphoreType.DMA] * (GB + OB),
    )
    def embed(tok_hbm, table_hbm, pe_hbm, out_hbm, tok_v, pe_v, table_s, *bufs):
        gs = bufs[:GB]
        os_ = bufs[GB:GB + OB]
        sgs = bufs[GB + OB:2 * GB + OB]
        sws = bufs[2 * GB + OB:]

        sub = lax.axis_index("s")
        wid = sub * 2 + lax.axis_index("c")
        base = wid * per_w

        rows_per_sub = TBL_PAD // 16
        pltpu.sync_copy(table_hbm.at[pl.ds(sub * rows_per_sub, rows_per_sub)],
                        table_s.at[pl.ds(sub * rows_per_sub, rows_per_sub)])
        pltpu.sync_copy(tok_hbm.at[pl.ds(base, per_w)], tok_v)
        pltpu.sync_copy(pe_hbm, pe_v.at[pl.ds(0, SEQ)])
        pltpu.sync_copy(pe_hbm.at[pl.ds(0, PE_EXT - SEQ)],
                        pe_v.at[pl.ds(SEQ, PE_EXT - SEQ)])
        plsc.subcore_barrier()

        def start_gather(c, s):
            pltpu.async_copy(
                table_s.at[tok_v.at[pl.ds(c * G, G)]], gs[s], sgs[s])

        def wait_gather(s):
            pltpu.make_async_copy(
                table_s.at[tok_v.at[pl.ds(0, G)]], gs[s], sgs[s]).wait()

        def start_write(c, s):
            pltpu.async_copy(
                os_[s], out_hbm.at[pl.ds(base + c * G, G)], sws[s])

        def wait_write(s):
            pltpu.make_async_copy(
                os_[s], out_hbm.at[pl.ds(base, G)], sws[s]).wait()

        def compute(c, s, so):
            pb = (c * G) % SEQ
            gv = gs[s]
            ov = os_[so]

            @plsc.parallel_loop(0, G, unroll=2)
            def _row(t):
                pr = pb + t
                slices = [pl.ds(j * LANES, LANES) for j in range(D_MODEL // LANES)]
                gvals = [gv[t, sl] for sl in slices]
                pvals = [pe_v[pr, sl] for sl in slices]
                for sl, gval, pval in zip(slices, gvals, pvals):
                    ov[t, sl] = gval + pval

        for c in range(GB):
            start_gather(c, c)
        for c in range(GB):
            wait_gather(c)
            if c >= OB:
                wait_write(c % OB)
            compute(c, c, c % OB)
            start_write(c, c % OB)
            start_gather(c + GB, c)

        @pl.loop(GB, nchunks - GB, step=GB)
        def _steady(c0):
            for k in range(GB):
                c = c0 + k
                wait_gather(k)
                wait_write(k % OB)
                compute(c, k, k % OB)
                start_write(c, k % OB)
                start_gather(c + GB, k)

        for k in range(GB):
            c = nchunks - GB + k
            wait_gather(k)
            wait_write(k % OB)
            compute(c, k, k % OB)
            start_write(c, k % OB)
        for s in range(OB):
            wait_write(s)

    return embed


def kernel(tokens, table, PE):
    batch, seq = tokens.shape
    n_tokens = batch * seq
    vocab = table.shape[0]
    table_p = jnp.concatenate(
        [table, jnp.zeros((TBL_PAD - vocab, table.shape[1]), table.dtype)])
    out = _build_kernel(n_tokens)(
        tokens.reshape(n_tokens), table_p, PE[:seq])
    return out.reshape(batch, seq, D_MODEL)
